# baseline (speedup 1.0000x reference)
import functools
import numpy as np
import jax
import jax.numpy as jnp
from jax import lax
from jax.experimental import pallas as pl
from jax.experimental.pallas import tpu as pltpu
from jax.experimental.pallas import tpu_sc as plsc

_R = 16
_NC = 2
_NS = 16


def _route_kernel(xh_ref, wq_ref, bq_ref, wk_ref, bk_ref, wv_ref, bv_ref,
                  pe_ref, q_out, wkv1_out, wkv2_out, c1_out, c2_out):
    h = pl.program_id(0)
    S, HD = xh_ref.shape[1], xh_ref.shape[2]
    P = pe_ref.shape[1]

    x = xh_ref[0]
    q = jnp.dot(x, wq_ref[0], preferred_element_type=jnp.float32) + bq_ref[0]
    kk = jnp.dot(x, wk_ref[0], preferred_element_type=jnp.float32) + bk_ref[0]
    vv = jnp.dot(x, wv_ref[0], preferred_element_type=jnp.float32) + bv_ref[0]

    logits = lax.dot_general(q, pe_ref[0], (((1,), (1,)), ((), ())),
                             preferred_element_type=jnp.float32)
    pid = lax.broadcasted_iota(jnp.int32, (S, P), 1)
    m1 = jnp.max(logits, axis=-1, keepdims=True)
    am1 = jnp.min(jnp.where(logits == m1, pid, P), axis=-1, keepdims=True)
    l2 = jnp.where(pid == am1, -jnp.inf, logits)
    m2 = jnp.max(l2, axis=-1, keepdims=True)
    am2 = jnp.min(jnp.where(l2 == m2, pid, P), axis=-1, keepdims=True)
    e2 = jnp.exp(m2 - m1)
    g1 = 1.0 / (1.0 + e2)
    g2 = e2 / (1.0 + e2)

    q_out[0] = q
    kv = jnp.concatenate([kk, vv], axis=1)
    wkv1_out[0] = g1 * kv
    wkv2_out[0] = g2 * kv

    hps = pl.num_programs(0) // _NC
    row = lax.broadcasted_iota(jnp.int32, (S, 1), 0) % _R
    base = (h % hps) * P * _R
    c1_out[0] = base + am1 * _R + row
    c2_out[0] = base + am2 * _R + row


def _attend_kernel(q_ref, st_ref, c1_ref, c2_ref, wot_ref, prev_ref, out_ref):
    h = pl.program_id(0)
    S, HD = q_ref.shape[1], q_ref.shape[2]
    PR = st_ref.shape[1]

    q = q_ref[0]
    st = st_ref[0]
    st_k = st[:, :HD]
    st_v = st[:, HD:]
    hps = pl.num_programs(0) // _NC
    base = (h % hps) * PR
    am1 = (c1_ref[0] - base) // _R
    am2 = (c2_ref[0] - base) // _R

    scores = lax.dot_general(q, st_k, (((1,), (1,)), ((), ())),
                             preferred_element_type=jnp.float32)
    scores = scores * (1.0 / np.sqrt(HD))
    cp = lax.broadcasted_iota(jnp.int32, (S, PR), 1) // _R
    sel = (cp == am1) | (cp == am2)
    sm = jnp.where(sel, scores, -jnp.inf)
    mx = jnp.max(sm, axis=-1, keepdims=True)
    prob = jnp.where(sel, jnp.exp(sm - mx), 0.0)
    aw = prob / jnp.sum(prob, axis=-1, keepdims=True)
    hv = jnp.dot(aw, st_v, preferred_element_type=jnp.float32)

    contrib = jnp.dot(hv, wot_ref[0], preferred_element_type=jnp.float32)

    @pl.when(h == 0)
    def _():
        out_ref[...] = jnp.broadcast_to(prev_ref[...], out_ref.shape)

    out_ref[...] += contrib


def _make_sc_scatter(Hh, S, HD, PR):
    rows_per_sc = (Hh // _NC) * S
    rpt = rows_per_sc // _NS
    local = (Hh // _NC) * PR
    slc = local // _NS
    n_sub = rpt // 128
    W = 2 * HD

    mesh = plsc.VectorSubcoreMesh(core_axis_name="c", subcore_axis_name="s",
                                  num_cores=_NC, num_subcores=_NS)

    @functools.partial(
        pl.kernel, mesh=mesh,
        out_type=jax.ShapeDtypeStruct((Hh * PR, W), jnp.float32),
        scratch_types=[
            pltpu.VMEM((rpt, W), jnp.float32),
            pltpu.VMEM((8, 128), jnp.int32),
            pltpu.VMEM_SHARED((local, W), jnp.float32),
        ])
    def scatter(wkv1, wkv2, c1, c2, zeros, st_o, rowbuf, idxbuf, st_sh):
        c = lax.axis_index("c")
        s = lax.axis_index("s")

        pltpu.sync_copy(zeros.at[pl.ds(s * slc, slc)],
                        st_sh.at[pl.ds(s * slc, slc)])
        plsc.subcore_barrier()

        base = c * rows_per_sc + s * rpt
        tid = c * _NS + s
        for src, idxsrc in ((wkv1, c1), (wkv2, c2)):
            pltpu.sync_copy(idxsrc.at[pl.ds(tid * 8, 8)], idxbuf)
            pltpu.sync_copy(src.at[pl.ds(base, rpt)], rowbuf)
            for j in range(n_sub):
                pltpu.sync_copy(rowbuf.at[pl.ds(j * 128, 128)],
                                st_sh.at[idxbuf.at[j]], add=True)
        plsc.subcore_barrier()

        out_base = c * local + s * slc
        pltpu.sync_copy(st_sh.at[pl.ds(s * slc, slc)],
                        st_o.at[pl.ds(out_base, slc)])

    return scatter


def kernel(x, Wq, bq, Wk, bk, Wv, bv, part_emb, Wo, bo):
    B, S, D = x.shape
    H, HD, _ = Wq.shape
    P = part_emb.shape[1]
    PR = P * _R
    Hh = H // 2

    xh = x.reshape(S, H, HD).transpose(1, 0, 2)
    wot = Wo.T.reshape(H, HD, D)
    bq3 = bq.reshape(H, 1, HD)
    bk3 = bk.reshape(H, 1, HD)
    bv3 = bv.reshape(H, 1, HD)
    bo2 = bo.reshape(1, D)

    head_spec = lambda shape: pl.BlockSpec(
        shape, lambda h: (h,) + (0,) * (len(shape) - 1))
    f32 = jnp.float32

    route_call = pl.pallas_call(
        _route_kernel,
        grid=(Hh,),
        in_specs=[
            head_spec((1, S, HD)),
            head_spec((1, HD, HD)), head_spec((1, 1, HD)),
            head_spec((1, HD, HD)), head_spec((1, 1, HD)),
            head_spec((1, HD, HD)), head_spec((1, 1, HD)),
            head_spec((1, P, HD)),
        ],
        out_specs=[head_spec((1, S, HD)),
                   head_spec((1, S, 2 * HD)), head_spec((1, S, 2 * HD)),
                   head_spec((1, S, 1)), head_spec((1, S, 1))],
        out_shape=[jax.ShapeDtypeStruct((Hh, S, HD), f32),
                   jax.ShapeDtypeStruct((Hh, S, 2 * HD), f32),
                   jax.ShapeDtypeStruct((Hh, S, 2 * HD), f32),
                   jax.ShapeDtypeStruct((Hh, S, 1), jnp.int32),
                   jax.ShapeDtypeStruct((Hh, S, 1), jnp.int32)],
        compiler_params=pltpu.CompilerParams(
            dimension_semantics=("arbitrary",)),
    )

    sc_scatter = _make_sc_scatter(Hh, S, HD, PR)
    zeros = jnp.zeros(((Hh // _NC) * PR, 2 * HD), f32)
    ntile = _NC * _NS
    grp = Hh * S // ntile // 128

    def make_attend(prev_shape):
        return pl.pallas_call(
            _attend_kernel,
            grid=(Hh,),
            in_specs=[
                head_spec((1, S, HD)),
                head_spec((1, PR, 2 * HD)),
                head_spec((1, S, 1)), head_spec((1, S, 1)),
                head_spec((1, HD, D)),
                pl.BlockSpec(prev_shape, lambda h: (0, 0)),
            ],
            out_specs=pl.BlockSpec((S, D), lambda h: (0, 0)),
            out_shape=jax.ShapeDtypeStruct((S, D), f32),
            compiler_params=pltpu.CompilerParams(
                dimension_semantics=("arbitrary",)),
        )

    out = bo2
    prev_shape = (1, D)
    for half in range(2):
        sl = slice(half * Hh, (half + 1) * Hh)
        q, wkv1, wkv2, c1, c2 = route_call(
            xh[sl], Wq[sl], bq3[sl], Wk[sl], bk3[sl], Wv[sl], bv3[sl],
            part_emb[sl])
        c1p = jnp.pad(c1.reshape(ntile, grp, 128),
                      ((0, 0), (0, 8 - grp), (0, 0))).reshape(ntile * 8, 128)
        c2p = jnp.pad(c2.reshape(ntile, grp, 128),
                      ((0, 0), (0, 8 - grp), (0, 0))).reshape(ntile * 8, 128)
        st = sc_scatter(wkv1.reshape(Hh * S, 2 * HD),
                        wkv2.reshape(Hh * S, 2 * HD), c1p, c2p, zeros)
        out = make_attend(prev_shape)(
            q, st.reshape(Hh, PR, 2 * HD), c1, c2, wot[sl], out)
        prev_shape = (S, D)

    return out.reshape(B, S, D)


# speedup vs baseline: 1.0737x; 1.0737x over previous
import functools
import numpy as np
import jax
import jax.numpy as jnp
from jax import lax
from jax.experimental import pallas as pl
from jax.experimental.pallas import tpu as pltpu
from jax.experimental.pallas import tpu_sc as plsc

_R = 16
_NC = 2
_NS = 16


def _route_kernel(xh_ref, wq_ref, bq_ref, wk_ref, bk_ref, wv_ref, bv_ref,
                  pe_ref, q_out, wkv1_out, wkv2_out, c1_out, c2_out):
    h = pl.program_id(0)
    S, HD = xh_ref.shape[1], xh_ref.shape[2]
    P = pe_ref.shape[1]

    x = xh_ref[0]
    q = jnp.dot(x, wq_ref[0], preferred_element_type=jnp.float32) + bq_ref[0]
    kk = jnp.dot(x, wk_ref[0], preferred_element_type=jnp.float32) + bk_ref[0]
    vv = jnp.dot(x, wv_ref[0], preferred_element_type=jnp.float32) + bv_ref[0]

    logits = lax.dot_general(q, pe_ref[0], (((1,), (1,)), ((), ())),
                             preferred_element_type=jnp.float32)
    pid = lax.broadcasted_iota(jnp.int32, (S, P), 1)
    m1 = jnp.max(logits, axis=-1, keepdims=True)
    am1 = jnp.min(jnp.where(logits == m1, pid, P), axis=-1, keepdims=True)
    l2 = jnp.where(pid == am1, -jnp.inf, logits)
    m2 = jnp.max(l2, axis=-1, keepdims=True)
    am2 = jnp.min(jnp.where(l2 == m2, pid, P), axis=-1, keepdims=True)
    e2 = jnp.exp(m2 - m1)
    g1 = 1.0 / (1.0 + e2)
    g2 = e2 / (1.0 + e2)

    q_out[0] = q
    kv = jnp.concatenate([kk, vv], axis=1)
    wkv1_out[0] = g1 * kv
    wkv2_out[0] = g2 * kv

    hps = pl.num_programs(0) // _NC
    row = lax.broadcasted_iota(jnp.int32, (S, 1), 0) % _R
    base = (h % hps) * P * _R
    c1_out[0] = base + am1 * _R + row
    c2_out[0] = base + am2 * _R + row


def _attend_kernel(q_ref, st_ref, c1_ref, c2_ref, wot_ref, bo_ref, out_ref):
    h = pl.program_id(0)
    S, HD = q_ref.shape[1], q_ref.shape[2]
    PR = st_ref.shape[1]

    q = q_ref[0]
    st = st_ref[0]
    st_k = st[:, :HD]
    st_v = st[:, HD:]
    hps = pl.num_programs(0) // _NC
    base = (h % hps) * PR
    am1 = (c1_ref[0] - base) // _R
    am2 = (c2_ref[0] - base) // _R

    scores = lax.dot_general(q, st_k, (((1,), (1,)), ((), ())),
                             preferred_element_type=jnp.float32)
    scores = scores * (1.0 / np.sqrt(HD))
    cp = lax.broadcasted_iota(jnp.int32, (S, PR), 1) // _R
    sel = (cp == am1) | (cp == am2)
    sm = jnp.where(sel, scores, -jnp.inf)
    mx = jnp.max(sm, axis=-1, keepdims=True)
    prob = jnp.where(sel, jnp.exp(sm - mx), 0.0)
    aw = prob / jnp.sum(prob, axis=-1, keepdims=True)
    hv = jnp.dot(aw, st_v, preferred_element_type=jnp.float32)

    contrib = jnp.dot(hv, wot_ref[0], preferred_element_type=jnp.float32)

    @pl.when(h == 0)
    def _():
        out_ref[...] = jnp.broadcast_to(bo_ref[...], out_ref.shape)

    out_ref[...] += contrib


def _make_sc_scatter(H, S, HD, PR):
    rows_per_sc = (H // _NC) * S
    rpt = rows_per_sc // _NS
    local = (H // _NC) * PR
    slc = local // _NS
    n_sub = rpt // 128
    W = 2 * HD

    mesh = plsc.VectorSubcoreMesh(core_axis_name="c", subcore_axis_name="s",
                                  num_cores=_NC, num_subcores=_NS)

    @functools.partial(
        pl.kernel, mesh=mesh,
        out_type=jax.ShapeDtypeStruct((H * PR, W), jnp.float32),
        scratch_types=[
            pltpu.VMEM((rpt, W), jnp.float32),
            pltpu.VMEM((8, 128), jnp.int32),
            pltpu.VMEM_SHARED((local, W), jnp.float32),
        ])
    def scatter(wkv1, wkv2, c1, c2, zeros, st_o, rowbuf, idxbuf, st_sh):
        c = lax.axis_index("c")
        s = lax.axis_index("s")

        pltpu.sync_copy(zeros.at[pl.ds(s * slc, slc)],
                        st_sh.at[pl.ds(s * slc, slc)])
        plsc.subcore_barrier()

        base = c * rows_per_sc + s * rpt
        tid = c * _NS + s
        for src, idxsrc in ((wkv1, c1), (wkv2, c2)):
            pltpu.sync_copy(idxsrc.at[pl.ds(tid * 8, 8)], idxbuf)
            pltpu.sync_copy(src.at[pl.ds(base, rpt)], rowbuf)
            for j in range(n_sub):
                pltpu.sync_copy(rowbuf.at[pl.ds(j * 128, 128)],
                                st_sh.at[idxbuf.at[j]], add=True)
        plsc.subcore_barrier()

        out_base = c * local + s * slc
        pltpu.sync_copy(st_sh.at[pl.ds(s * slc, slc)],
                        st_o.at[pl.ds(out_base, slc)])

    return scatter


def kernel(x, Wq, bq, Wk, bk, Wv, bv, part_emb, Wo, bo):
    B, S, D = x.shape
    H, HD, _ = Wq.shape
    P = part_emb.shape[1]
    PR = P * _R

    xh = x.reshape(S, H, HD).transpose(1, 0, 2)
    wot = Wo.T.reshape(H, HD, D)
    bq3 = bq.reshape(H, 1, HD)
    bk3 = bk.reshape(H, 1, HD)
    bv3 = bv.reshape(H, 1, HD)
    bo2 = bo.reshape(1, D)

    head_spec = lambda shape: pl.BlockSpec(
        shape, lambda h: (h,) + (0,) * (len(shape) - 1))
    f32 = jnp.float32

    q, wkv1, wkv2, c1, c2 = pl.pallas_call(
        _route_kernel,
        grid=(H,),
        in_specs=[
            head_spec((1, S, HD)),
            head_spec((1, HD, HD)), head_spec((1, 1, HD)),
            head_spec((1, HD, HD)), head_spec((1, 1, HD)),
            head_spec((1, HD, HD)), head_spec((1, 1, HD)),
            head_spec((1, P, HD)),
        ],
        out_specs=[head_spec((1, S, HD)),
                   head_spec((1, S, 2 * HD)), head_spec((1, S, 2 * HD)),
                   head_spec((1, S, 1)), head_spec((1, S, 1))],
        out_shape=[jax.ShapeDtypeStruct((H, S, HD), f32),
                   jax.ShapeDtypeStruct((H, S, 2 * HD), f32),
                   jax.ShapeDtypeStruct((H, S, 2 * HD), f32),
                   jax.ShapeDtypeStruct((H, S, 1), jnp.int32),
                   jax.ShapeDtypeStruct((H, S, 1), jnp.int32)],
        compiler_params=pltpu.CompilerParams(
            dimension_semantics=("arbitrary",)),
    )(xh, Wq, bq3, Wk, bk3, Wv, bv3, part_emb)

    sc_scatter = _make_sc_scatter(H, S, HD, PR)
    zeros = jnp.zeros(((H // _NC) * PR, 2 * HD), f32)
    ntile = _NC * _NS
    grp = H * S // ntile // 128
    c1p = jnp.pad(c1.reshape(ntile, grp, 128),
                  ((0, 0), (0, 8 - grp), (0, 0))).reshape(ntile * 8, 128)
    c2p = jnp.pad(c2.reshape(ntile, grp, 128),
                  ((0, 0), (0, 8 - grp), (0, 0))).reshape(ntile * 8, 128)
    st = sc_scatter(wkv1.reshape(H * S, 2 * HD), wkv2.reshape(H * S, 2 * HD),
                    c1p, c2p, zeros)

    out = pl.pallas_call(
        _attend_kernel,
        grid=(H,),
        in_specs=[
            head_spec((1, S, HD)),
            head_spec((1, PR, 2 * HD)),
            head_spec((1, S, 1)), head_spec((1, S, 1)),
            head_spec((1, HD, D)),
            pl.BlockSpec((1, D), lambda h: (0, 0)),
        ],
        out_specs=pl.BlockSpec((S, D), lambda h: (0, 0)),
        out_shape=jax.ShapeDtypeStruct((S, D), f32),
        compiler_params=pltpu.CompilerParams(
            dimension_semantics=("arbitrary",)),
    )(q, st.reshape(H, PR, 2 * HD), c1, c2, wot, bo2)
    return out.reshape(B, S, D)


# speedup vs baseline: 1.0751x; 1.0013x over previous
import functools
import numpy as np
import jax
import jax.numpy as jnp
from jax import lax
from jax.experimental import pallas as pl
from jax.experimental.pallas import tpu as pltpu
from jax.experimental.pallas import tpu_sc as plsc

_R = 16
_NC = 2
_NS = 16


def _route_kernel(xh_ref, wq_ref, bq_ref, wk_ref, bk_ref, wv_ref, bv_ref,
                  pe_ref, q_out, wkv1_out, wkv2_out, c1_out, c2_out):
    h = pl.program_id(0)
    S, HD = xh_ref.shape[1], xh_ref.shape[2]
    P = pe_ref.shape[1]

    x = xh_ref[0]
    q = jnp.dot(x, wq_ref[0], preferred_element_type=jnp.float32) + bq_ref[0]
    kk = jnp.dot(x, wk_ref[0], preferred_element_type=jnp.float32) + bk_ref[0]
    vv = jnp.dot(x, wv_ref[0], preferred_element_type=jnp.float32) + bv_ref[0]

    logits = lax.dot_general(q, pe_ref[0], (((1,), (1,)), ((), ())),
                             preferred_element_type=jnp.float32)
    pid = lax.broadcasted_iota(jnp.int32, (S, P), 1)
    m1 = jnp.max(logits, axis=-1, keepdims=True)
    am1 = jnp.min(jnp.where(logits == m1, pid, P), axis=-1, keepdims=True)
    l2 = jnp.where(pid == am1, -jnp.inf, logits)
    m2 = jnp.max(l2, axis=-1, keepdims=True)
    am2 = jnp.min(jnp.where(l2 == m2, pid, P), axis=-1, keepdims=True)
    e2 = jnp.exp(m2 - m1)
    g1 = 1.0 / (1.0 + e2)
    g2 = e2 / (1.0 + e2)

    q_out[0] = q
    kv = jnp.concatenate([kk, vv], axis=1)
    wkv1_out[0] = g1 * kv
    wkv2_out[0] = g2 * kv

    hps = pl.num_programs(0) // _NC
    row = lax.broadcasted_iota(jnp.int32, (S, 1), 0) % _R
    base = (h % hps) * P * _R
    c1_out[0] = base + am1 * _R + row
    c2_out[0] = base + am2 * _R + row


def _attend_kernel(q_ref, st_ref, c1_ref, c2_ref, wot_ref, bo_ref, out_ref):
    h = pl.program_id(0)
    S, HD = q_ref.shape[1], q_ref.shape[2]
    PR = st_ref.shape[1]

    q = q_ref[0]
    st = st_ref[0]
    st_k = st[:, :HD]
    st_v = st[:, HD:]
    hps = pl.num_programs(0) // _NC
    base = (h % hps) * PR
    am1 = (c1_ref[0] - base) // _R
    am2 = (c2_ref[0] - base) // _R

    scores = lax.dot_general(q, st_k, (((1,), (1,)), ((), ())),
                             preferred_element_type=jnp.float32)
    scores = scores * (1.0 / np.sqrt(HD))
    cp = lax.broadcasted_iota(jnp.int32, (S, PR), 1) // _R
    sel = (cp == am1) | (cp == am2)
    sm = jnp.where(sel, scores, -jnp.inf)
    mx = jnp.max(sm, axis=-1, keepdims=True)
    prob = jnp.where(sel, jnp.exp(sm - mx), 0.0)
    aw = prob / jnp.sum(prob, axis=-1, keepdims=True)
    hv = jnp.dot(aw, st_v, preferred_element_type=jnp.float32)

    contrib = jnp.dot(hv, wot_ref[0], preferred_element_type=jnp.float32)

    @pl.when(h == 0)
    def _():
        out_ref[...] = jnp.broadcast_to(bo_ref[...], out_ref.shape)

    out_ref[...] += contrib


def _make_sc_scatter(H, S, HD, PR):
    rows_per_sc = (H // _NC) * S
    rpt = rows_per_sc // _NS
    local = (H // _NC) * PR
    slc = local // _NS
    n_sub = rpt // 128
    W = 2 * HD

    mesh = plsc.VectorSubcoreMesh(core_axis_name="c", subcore_axis_name="s",
                                  num_cores=_NC, num_subcores=_NS)

    @functools.partial(
        pl.kernel, mesh=mesh,
        out_type=jax.ShapeDtypeStruct((H * PR, W), jnp.float32),
        scratch_types=[
            pltpu.VMEM((rpt, W), jnp.float32),
            pltpu.VMEM((8, 128), jnp.int32),
            pltpu.VMEM_SHARED((local, W), jnp.float32),
        ])
    def scatter(wkv1, wkv2, c1, c2, zeros, st_o, rowbuf, idxbuf, st_sh):
        c = lax.axis_index("c")
        s = lax.axis_index("s")

        pltpu.sync_copy(zeros.at[pl.ds(s * slc, slc)],
                        st_sh.at[pl.ds(s * slc, slc)])
        plsc.subcore_barrier()

        base = c * rows_per_sc + s * rpt
        tid = c * _NS + s
        for src, idxsrc in ((wkv1, c1), (wkv2, c2)):
            pltpu.sync_copy(idxsrc.at[pl.ds(tid * 8, 8)], idxbuf)
            pltpu.sync_copy(src.at[pl.ds(base, rpt)], rowbuf)
            for j in range(n_sub):
                pltpu.sync_copy(rowbuf.at[pl.ds(j * 128, 128)],
                                st_sh.at[idxbuf.at[j]], add=True)
        plsc.subcore_barrier()

        out_base = c * local + s * slc
        pltpu.sync_copy(st_sh.at[pl.ds(s * slc, slc)],
                        st_o.at[pl.ds(out_base, slc)])

    return scatter


def kernel(x, Wq, bq, Wk, bk, Wv, bv, part_emb, Wo, bo):
    B, S, D = x.shape
    H, HD, _ = Wq.shape
    P = part_emb.shape[1]
    PR = P * _R

    xh = x.reshape(S, H, HD).transpose(1, 0, 2)
    wot = Wo.T.reshape(H, HD, D)
    bq3 = bq.reshape(H, 1, HD)
    bk3 = bk.reshape(H, 1, HD)
    bv3 = bv.reshape(H, 1, HD)
    bo2 = bo.reshape(1, D)

    head_spec = lambda shape: pl.BlockSpec(
        shape, lambda h: (h,) + (0,) * (len(shape) - 1))
    f32 = jnp.float32

    q, wkv1, wkv2, c1, c2 = pl.pallas_call(
        _route_kernel,
        grid=(H,),
        in_specs=[
            head_spec((1, S, HD)),
            head_spec((1, HD, HD)), head_spec((1, 1, HD)),
            head_spec((1, HD, HD)), head_spec((1, 1, HD)),
            head_spec((1, HD, HD)), head_spec((1, 1, HD)),
            head_spec((1, P, HD)),
        ],
        out_specs=[head_spec((1, S, HD)),
                   head_spec((1, S, 2 * HD)), head_spec((1, S, 2 * HD)),
                   head_spec((1, S, 1)), head_spec((1, S, 1))],
        out_shape=[jax.ShapeDtypeStruct((H, S, HD), f32),
                   jax.ShapeDtypeStruct((H, S, 2 * HD), f32),
                   jax.ShapeDtypeStruct((H, S, 2 * HD), f32),
                   jax.ShapeDtypeStruct((H, S, 1), jnp.int32),
                   jax.ShapeDtypeStruct((H, S, 1), jnp.int32)],
        compiler_params=pltpu.CompilerParams(
            dimension_semantics=("parallel",)),
    )(xh, Wq, bq3, Wk, bk3, Wv, bv3, part_emb)

    sc_scatter = _make_sc_scatter(H, S, HD, PR)
    zeros = jnp.zeros(((H // _NC) * PR, 2 * HD), f32)
    ntile = _NC * _NS
    grp = H * S // ntile // 128
    c1p = jnp.pad(c1.reshape(ntile, grp, 128),
                  ((0, 0), (0, 8 - grp), (0, 0))).reshape(ntile * 8, 128)
    c2p = jnp.pad(c2.reshape(ntile, grp, 128),
                  ((0, 0), (0, 8 - grp), (0, 0))).reshape(ntile * 8, 128)
    st = sc_scatter(wkv1.reshape(H * S, 2 * HD), wkv2.reshape(H * S, 2 * HD),
                    c1p, c2p, zeros)

    out = pl.pallas_call(
        _attend_kernel,
        grid=(H,),
        in_specs=[
            head_spec((1, S, HD)),
            head_spec((1, PR, 2 * HD)),
            head_spec((1, S, 1)), head_spec((1, S, 1)),
            head_spec((1, HD, D)),
            pl.BlockSpec((1, D), lambda h: (0, 0)),
        ],
        out_specs=pl.BlockSpec((S, D), lambda h: (0, 0)),
        out_shape=jax.ShapeDtypeStruct((S, D), f32),
        compiler_params=pltpu.CompilerParams(
            dimension_semantics=("arbitrary",)),
    )(q, st.reshape(H, PR, 2 * HD), c1, c2, wot, bo2)
    return out.reshape(B, S, D)
